# baseline (speedup 1.0000x reference)
import jax
import jax.numpy as jnp
from jax.experimental import pallas as pl
from jax.experimental.pallas import tpu as pltpu

_BS = 2048


def _add_body(x_ref, pe_ref, o_ref):
    o_ref[...] = x_ref[...] + pe_ref[...][None]


def kernel(x, pe_table):
    B, S, D = x.shape
    BD = D // 2
    grid = (S // _BS, 2, B)
    return pl.pallas_call(
        _add_body,
        grid=grid,
        in_specs=[
            pl.BlockSpec((1, _BS, BD), lambda s, d, b: (b, s, d)),
            pl.BlockSpec((_BS, BD), lambda s, d, b: (s, d)),
        ],
        out_specs=pl.BlockSpec((1, _BS, BD), lambda s, d, b: (b, s, d)),
        out_shape=jax.ShapeDtypeStruct((B, S, D), x.dtype),
        compiler_params=pltpu.CompilerParams(
            dimension_semantics=("arbitrary", "arbitrary", "arbitrary"),
        ),
    )(x, pe_table)
